# baseline (speedup 1.0000x reference)
import jax
import jax.numpy as jnp
from jax.experimental import pallas as pl
from jax.experimental.pallas import tpu as pltpu

_N = 50000
_D = 512
_G = 128
_C = 128
_B = 400
_NB = _N // _B


def _ff_kernel(bounds_ref, x_ref, batch_ref, w_enc_ref, b_enc_ref,
               w_out_ref, b_out_ref, out_ref, pooled_ref):
    i = pl.program_id(0)
    nb = pl.num_programs(0)

    @pl.when(i == 0)
    def _init():
        pooled_ref[...] = jnp.full_like(pooled_ref, -jnp.inf)

    h = jnp.maximum(
        jnp.dot(x_ref[...], w_enc_ref[...], preferred_element_type=jnp.float32)
        + b_enc_ref[...],
        0.0,
    )

    bb = batch_ref[...]
    s0 = bounds_ref[i, 0]
    s1 = bounds_ref[i, 1]

    def body(g, carry):
        m = jnp.max(jnp.where(bb == g, h, -jnp.inf), axis=0, keepdims=True)
        cur = pooled_ref[pl.ds(g, 1), :]
        pooled_ref[pl.ds(g, 1), :] = jnp.maximum(cur, m)
        return carry

    jax.lax.fori_loop(s0, s1 + 1, body, 0)

    @pl.when(i == nb - 1)
    def _finish():
        logits = (
            jnp.dot(pooled_ref[...], w_out_ref[...],
                    preferred_element_type=jnp.float32)
            + b_out_ref[...]
        )
        mx = jnp.max(logits, axis=1, keepdims=True)
        sh = logits - mx
        lse = jnp.log(jnp.sum(jnp.exp(sh), axis=1, keepdims=True))
        out_ref[...] = sh - lse


def kernel(x, batch, W_enc, b_enc, W_out, b_out):
    batch = batch.astype(jnp.int32)
    batch_col = batch.reshape(_N, 1)
    starts = jnp.arange(_NB, dtype=jnp.int32) * _B
    bounds = jnp.stack([batch[starts], batch[starts + _B - 1]], axis=1)

    grid_spec = pltpu.PrefetchScalarGridSpec(
        num_scalar_prefetch=1,
        grid=(_NB,),
        in_specs=[
            pl.BlockSpec((_B, _D), lambda i, b: (i, 0)),
            pl.BlockSpec((_B, 1), lambda i, b: (i, 0)),
            pl.BlockSpec((_D, _D), lambda i, b: (0, 0)),
            pl.BlockSpec((1, _D), lambda i, b: (0, 0)),
            pl.BlockSpec((_D, _C), lambda i, b: (0, 0)),
            pl.BlockSpec((1, _C), lambda i, b: (0, 0)),
        ],
        out_specs=pl.BlockSpec((_G, _C), lambda i, b: (0, 0)),
        scratch_shapes=[pltpu.VMEM((_G, _D), jnp.float32)],
    )

    return pl.pallas_call(
        _ff_kernel,
        grid_spec=grid_spec,
        out_shape=jax.ShapeDtypeStruct((_G, _C), jnp.float32),
        compiler_params=pltpu.CompilerParams(
            dimension_semantics=("arbitrary",),
        ),
    )(bounds, x, batch_col, W_enc, b_enc.reshape(1, _D), W_out,
      b_out.reshape(1, _C))


# speedup vs baseline: 1.0629x; 1.0629x over previous
import jax
import jax.numpy as jnp
from jax.experimental import pallas as pl
from jax.experimental.pallas import tpu as pltpu

_N = 50000
_D = 512
_G = 128
_C = 128
_B = 1000
_NB = _N // _B
_W = 256


def _ff_kernel(bounds_ref, segoff_ref, x_ref, batch_ref, w_enc_ref, b_enc_ref,
               w_out_ref, b_out_ref, out_ref, pooled_ref, p_ref):
    i = pl.program_id(0)
    nb = pl.num_programs(0)

    @pl.when(i == 0)
    def _init():
        pooled_ref[...] = jnp.full_like(pooled_ref, -jnp.inf)

    p_ref[...] = jnp.dot(x_ref[...], w_enc_ref[...],
                         preferred_element_type=jnp.float32)

    row0 = i * _B
    s0 = bounds_ref[i, 0]
    s1 = bounds_ref[i, 1]

    def seg_body(g, carry):
        lo = jnp.maximum(segoff_ref[g] - row0, 0)
        hi = jnp.minimum(segoff_ref[g + 1] - row0, _B)
        base = (lo // 8) * 8
        nch = (hi - base + _W - 1) // _W

        def chunk_body(c, acc):
            off = jnp.minimum(base + c * _W, _B - _W)
            rows = p_ref[pl.ds(off, _W), :]
            bbw = batch_ref[pl.ds(off, _W), :]
            m = jnp.where(bbw == g, rows, -jnp.inf)
            return jnp.maximum(acc, jnp.max(m, axis=0, keepdims=True))

        acc = jax.lax.fori_loop(
            0, nch, chunk_body,
            jnp.full((1, _D), -jnp.inf, dtype=jnp.float32))
        cur = pooled_ref[pl.ds(g, 1), :]
        pooled_ref[pl.ds(g, 1), :] = jnp.maximum(cur, acc)
        return carry

    jax.lax.fori_loop(s0, s1 + 1, seg_body, 0)

    @pl.when(i == nb - 1)
    def _finish():
        pooled = jnp.maximum(pooled_ref[...] + b_enc_ref[...], 0.0)
        logits = (
            jnp.dot(pooled, w_out_ref[...], preferred_element_type=jnp.float32)
            + b_out_ref[...]
        )
        mx = jnp.max(logits, axis=1, keepdims=True)
        sh = logits - mx
        lse = jnp.log(jnp.sum(jnp.exp(sh), axis=1, keepdims=True))
        out_ref[...] = sh - lse


def kernel(x, batch, W_enc, b_enc, W_out, b_out):
    batch = batch.astype(jnp.int32)
    batch_col = batch.reshape(_N, 1)
    starts = jnp.arange(_NB, dtype=jnp.int32) * _B
    bounds = jnp.stack([batch[starts], batch[starts + _B - 1]], axis=1)
    segoff = jnp.searchsorted(
        batch, jnp.arange(_G + 1, dtype=jnp.int32)).astype(jnp.int32)

    grid_spec = pltpu.PrefetchScalarGridSpec(
        num_scalar_prefetch=2,
        grid=(_NB,),
        in_specs=[
            pl.BlockSpec((_B, _D), lambda i, b, s: (i, 0)),
            pl.BlockSpec((_B, 1), lambda i, b, s: (i, 0)),
            pl.BlockSpec((_D, _D), lambda i, b, s: (0, 0)),
            pl.BlockSpec((1, _D), lambda i, b, s: (0, 0)),
            pl.BlockSpec((_D, _C), lambda i, b, s: (0, 0)),
            pl.BlockSpec((1, _C), lambda i, b, s: (0, 0)),
        ],
        out_specs=pl.BlockSpec((_G, _C), lambda i, b, s: (0, 0)),
        scratch_shapes=[pltpu.VMEM((_G, _D), jnp.float32),
                        pltpu.VMEM((_B, _D), jnp.float32)],
    )

    return pl.pallas_call(
        _ff_kernel,
        grid_spec=grid_spec,
        out_shape=jax.ShapeDtypeStruct((_G, _C), jnp.float32),
        compiler_params=pltpu.CompilerParams(
            dimension_semantics=("arbitrary",),
        ),
    )(bounds, segoff, x, batch_col, W_enc, b_enc.reshape(1, _D), W_out,
      b_out.reshape(1, _C))


# speedup vs baseline: 1.3714x; 1.2902x over previous
import jax
import jax.numpy as jnp
from jax.experimental import pallas as pl
from jax.experimental.pallas import tpu as pltpu

_N = 50000
_D = 512
_G = 128
_C = 128
_B = 1000
_NB = _N // _B


def _ff_kernel(bounds_ref, x_ref, batch_ref, w_enc_ref, b_enc_ref,
               w_out_ref, b_out_ref, out_ref, pooled_ref):
    i = pl.program_id(0)
    nb = pl.num_programs(0)

    @pl.when(i == 0)
    def _init():
        pooled_ref[...] = jnp.full_like(pooled_ref, -jnp.inf)

    h = jnp.dot(x_ref[...], w_enc_ref[...], preferred_element_type=jnp.float32)

    bb = batch_ref[...]
    s0 = bounds_ref[i, 0]
    s1 = bounds_ref[i, 1]

    def body(g, carry):
        m = jnp.max(jnp.where(bb == g, h, -jnp.inf), axis=0, keepdims=True)
        cur = pooled_ref[pl.ds(g, 1), :]
        pooled_ref[pl.ds(g, 1), :] = jnp.maximum(cur, m)
        return carry

    jax.lax.fori_loop(s0, s1 + 1, body, 0)

    @pl.when(i == nb - 1)
    def _finish():
        pooled = jnp.maximum(pooled_ref[...] + b_enc_ref[...], 0.0)
        logits = (
            jnp.dot(pooled, w_out_ref[...],
                    preferred_element_type=jnp.float32)
            + b_out_ref[...]
        )
        mx = jnp.max(logits, axis=1, keepdims=True)
        sh = logits - mx
        lse = jnp.log(jnp.sum(jnp.exp(sh), axis=1, keepdims=True))
        out_ref[...] = sh - lse


def kernel(x, batch, W_enc, b_enc, W_out, b_out):
    batch = batch.astype(jnp.int32)
    batch_col = batch.reshape(_N, 1)
    starts = jnp.arange(_NB, dtype=jnp.int32) * _B
    bounds = jnp.stack([batch[starts], batch[starts + _B - 1]], axis=1)

    grid_spec = pltpu.PrefetchScalarGridSpec(
        num_scalar_prefetch=1,
        grid=(_NB,),
        in_specs=[
            pl.BlockSpec((_B, _D), lambda i, b: (i, 0)),
            pl.BlockSpec((_B, 1), lambda i, b: (i, 0)),
            pl.BlockSpec((_D, _D), lambda i, b: (0, 0)),
            pl.BlockSpec((1, _D), lambda i, b: (0, 0)),
            pl.BlockSpec((_D, _C), lambda i, b: (0, 0)),
            pl.BlockSpec((1, _C), lambda i, b: (0, 0)),
        ],
        out_specs=pl.BlockSpec((_G, _C), lambda i, b: (0, 0)),
        scratch_shapes=[pltpu.VMEM((_G, _D), jnp.float32)],
    )

    return pl.pallas_call(
        _ff_kernel,
        grid_spec=grid_spec,
        out_shape=jax.ShapeDtypeStruct((_G, _C), jnp.float32),
        compiler_params=pltpu.CompilerParams(
            dimension_semantics=("arbitrary",),
        ),
    )(bounds, x, batch_col, W_enc, b_enc.reshape(1, _D), W_out,
      b_out.reshape(1, _C))


# speedup vs baseline: 1.3797x; 1.0061x over previous
import jax
import jax.numpy as jnp
from jax.experimental import pallas as pl
from jax.experimental.pallas import tpu as pltpu

_N = 50000
_D = 512
_G = 128
_C = 128
_B = 1000
_NB = _N // _B
_NC = 5
_W = _B // _NC


def _ff_kernel(bounds_ref, x_ref, batch_ref, w_enc_ref, b_enc_ref,
               w_out_ref, b_out_ref, out_ref, pooled_ref):
    i = pl.program_id(0)
    nb = pl.num_programs(0)

    @pl.when(i == 0)
    def _init():
        pooled_ref[...] = jnp.full_like(pooled_ref, -jnp.inf)

    h = jnp.dot(x_ref[...], w_enc_ref[...], preferred_element_type=jnp.float32)

    bb = batch_ref[...]
    for c in range(_NC):
        hc = h[c * _W:(c + 1) * _W, :]
        bbc = bb[c * _W:(c + 1) * _W, :]
        s0 = bounds_ref[i * _NC + c, 0]
        s1 = bounds_ref[i * _NC + c, 1]

        @pl.when(s0 == s1)
        def _pure(hc=hc, s0=s0):
            m = jnp.max(hc, axis=0, keepdims=True)
            cur = pooled_ref[pl.ds(s0, 1), :]
            pooled_ref[pl.ds(s0, 1), :] = jnp.maximum(cur, m)

        @pl.when(s0 != s1)
        def _mixed(hc=hc, bbc=bbc, s0=s0, s1=s1):
            def body(g, carry):
                m = jnp.max(jnp.where(bbc == g, hc, -jnp.inf), axis=0,
                            keepdims=True)
                cur = pooled_ref[pl.ds(g, 1), :]
                pooled_ref[pl.ds(g, 1), :] = jnp.maximum(cur, m)
                return carry

            jax.lax.fori_loop(s0, s1 + 1, body, 0)

    @pl.when(i == nb - 1)
    def _finish():
        pooled = jnp.maximum(pooled_ref[...] + b_enc_ref[...], 0.0)
        logits = (
            jnp.dot(pooled, w_out_ref[...],
                    preferred_element_type=jnp.float32)
            + b_out_ref[...]
        )
        mx = jnp.max(logits, axis=1, keepdims=True)
        sh = logits - mx
        lse = jnp.log(jnp.sum(jnp.exp(sh), axis=1, keepdims=True))
        out_ref[...] = sh - lse


def kernel(x, batch, W_enc, b_enc, W_out, b_out):
    batch = batch.astype(jnp.int32)
    batch_col = batch.reshape(_N, 1)
    starts = jnp.arange(_NB * _NC, dtype=jnp.int32) * _W
    bounds = jnp.stack([batch[starts], batch[starts + _W - 1]], axis=1)

    grid_spec = pltpu.PrefetchScalarGridSpec(
        num_scalar_prefetch=1,
        grid=(_NB,),
        in_specs=[
            pl.BlockSpec((_B, _D), lambda i, b: (i, 0)),
            pl.BlockSpec((_B, 1), lambda i, b: (i, 0)),
            pl.BlockSpec((_D, _D), lambda i, b: (0, 0)),
            pl.BlockSpec((1, _D), lambda i, b: (0, 0)),
            pl.BlockSpec((_D, _C), lambda i, b: (0, 0)),
            pl.BlockSpec((1, _C), lambda i, b: (0, 0)),
        ],
        out_specs=pl.BlockSpec((_G, _C), lambda i, b: (0, 0)),
        scratch_shapes=[pltpu.VMEM((_G, _D), jnp.float32)],
    )

    return pl.pallas_call(
        _ff_kernel,
        grid_spec=grid_spec,
        out_shape=jax.ShapeDtypeStruct((_G, _C), jnp.float32),
        compiler_params=pltpu.CompilerParams(
            dimension_semantics=("arbitrary",),
        ),
    )(bounds, x, batch_col, W_enc, b_enc.reshape(1, _D), W_out,
      b_out.reshape(1, _C))
